# baseline (speedup 1.0000x reference)
import jax
import jax.numpy as jnp
from jax.experimental import pallas as pl
from jax.experimental.pallas import tpu as pltpu

ROWS, COLS, NF = 64, 128, 32
BATCH = 4096
SB = 256


def _bmu_kernel(x_ref, c_ref, out_ref):
    c = c_ref[...]
    c2 = jnp.sum(c * c, axis=1).reshape(1, -1)
    for i in range(BATCH // SB):
        xs = x_ref[pl.ds(i * SB, SB), :]
        x2 = jnp.sum(xs * xs, axis=1, keepdims=True)
        dot2 = jax.lax.dot_general(
            2.0 * xs, c, (((1,), (1,)), ((), ())),
            preferred_element_type=jnp.float32)
        d2 = (x2 + c2) - dot2
        idx = jnp.argmin(d2, axis=1).astype(jnp.int32)
        out_ref[pl.ds(i * SB, SB), :] = jnp.concatenate(
            [(idx // COLS)[:, None], (idx % COLS)[:, None]], axis=1)


def kernel(xb, weights):
    codebook = weights.reshape(-1, NF)
    return pl.pallas_call(
        _bmu_kernel,
        out_shape=jax.ShapeDtypeStruct((BATCH, 2), jnp.int32),
    )(xb, codebook)


# speedup vs baseline: 1.0163x; 1.0163x over previous
import jax
import jax.numpy as jnp
from jax.experimental import pallas as pl
from jax.experimental.pallas import tpu as pltpu

ROWS, COLS, NF = 64, 128, 32
BATCH = 4096
SB = 1024


def _bmu_kernel(x_ref, c_ref, out_ref):
    c = c_ref[...]
    c2 = jnp.sum(c * c, axis=1).reshape(1, -1)
    for i in range(BATCH // SB):
        xs = x_ref[pl.ds(i * SB, SB), :]
        x2 = jnp.sum(xs * xs, axis=1, keepdims=True)
        dot2 = jax.lax.dot_general(
            2.0 * xs, c, (((1,), (1,)), ((), ())),
            preferred_element_type=jnp.float32)
        d2 = (x2 + c2) - dot2
        idx = jnp.argmin(d2, axis=1).astype(jnp.int32)
        out_ref[pl.ds(i * SB, SB), :] = jnp.concatenate(
            [(idx // COLS)[:, None], (idx % COLS)[:, None]], axis=1)


def kernel(xb, weights):
    codebook = weights.reshape(-1, NF)
    return pl.pallas_call(
        _bmu_kernel,
        out_shape=jax.ShapeDtypeStruct((BATCH, 2), jnp.int32),
    )(xb, codebook)
